# baseline (speedup 1.0000x reference)
import functools

import jax
import jax.numpy as jnp
from jax import lax
from jax.experimental import pallas as pl
from jax.experimental.pallas import tpu as pltpu
from jax.experimental.pallas import tpu_sc as plsc

N = 10000
E = 320000
D = 128
K = 10
ALPHA = 0.1

NW = 32
N2 = 10240
CHUNK = 128
NCH = 80
EPW = CHUNK * NCH
E2 = NW * EPW
RPT = N2 // 16
NBUF = 2
NI = 2 * NBUF

_mesh = plsc.VectorSubcoreMesh(core_axis_name="c", subcore_axis_name="s")


@functools.partial(
    pl.kernel,
    mesh=_mesh,
    out_type=jax.ShapeDtypeStruct((2, 2, N2), jnp.float32),
    scratch_types=[
        pltpu.VMEM((NCH, 2, CHUNK), jnp.int32),
        pltpu.VMEM((CHUNK,), jnp.float32),
        pltpu.VMEM_SHARED((N2,), jnp.float32),
        pltpu.VMEM_SHARED((N2,), jnp.float32),
    ],
)
def _deg_kernel(ei_hbm, zd_hbm, degout_hbm,
                idx_v, ones_v, dsrc_sh, ddst_sh):
    c = lax.axis_index("c")
    s = lax.axis_index("s")
    w = s * 2 + c
    for i in range(CHUNK // 16):
        ones_v[pl.ds(i * 16, 16)] = jnp.ones((16,), jnp.float32)
    pltpu.sync_copy(zd_hbm.at[0, pl.ds(s * RPT, RPT)],
                    dsrc_sh.at[pl.ds(s * RPT, RPT)])
    pltpu.sync_copy(zd_hbm.at[1, pl.ds(s * RPT, RPT)],
                    ddst_sh.at[pl.ds(s * RPT, RPT)])
    plsc.subcore_barrier()

    pltpu.sync_copy(ei_hbm.at[w], idx_v)

    def body(j, carry):
        pltpu.sync_copy(ones_v, dsrc_sh.at[idx_v.at[j, 0]], add=True)
        pltpu.sync_copy(ones_v, ddst_sh.at[idx_v.at[j, 1]], add=True)
        return carry

    lax.fori_loop(0, NCH, body, 0)
    plsc.subcore_barrier()

    pltpu.sync_copy(dsrc_sh.at[pl.ds(s * RPT, RPT)],
                    degout_hbm.at[c, 0, pl.ds(s * RPT, RPT)])
    pltpu.sync_copy(ddst_sh.at[pl.ds(s * RPT, RPT)],
                    degout_hbm.at[c, 1, pl.ds(s * RPT, RPT)])


@functools.partial(
    pl.kernel,
    mesh=_mesh,
    out_type=jax.ShapeDtypeStruct((2, N2, D), jnp.float32),
    scratch_types=[
        pltpu.VMEM((NI, 2, CHUNK), jnp.int32),
        pltpu.VMEM((NBUF, CHUNK, D), jnp.float32),
        pltpu.VMEM_SHARED((N2, D), jnp.float32),
    ]
    + [pltpu.SemaphoreType.DMA] * (NI + NBUF),
)
def _prop_kernel(g_hbm, ei_hbm, znd_hbm, pout_hbm,
                 ring, rowbuf, agg_sh, *sems):
    isem = sems[:NI]
    gsem = sems[NI:]
    c = lax.axis_index("c")
    s = lax.axis_index("s")
    w = s * 2 + c
    for b in range(NI):
        pltpu.async_copy(ei_hbm.at[w, b], ring.at[b], isem[b])
    pltpu.sync_copy(znd_hbm.at[pl.ds(s * RPT, RPT)],
                    agg_sh.at[pl.ds(s * RPT, RPT)])
    plsc.subcore_barrier()

    for b in range(NBUF):
        pltpu.make_async_copy(ei_hbm.at[w, b], ring.at[b], isem[b]).wait()
        pltpu.async_copy(g_hbm.at[ring.at[b, 0]], rowbuf.at[b], gsem[b])

    def group_body(gi, carry):
        for b in range(NI):
            j = gi * NI + b
            rb = b % NBUF
            sbn = (b + NBUF) % NI
            pltpu.make_async_copy(
                g_hbm.at[ring.at[b, 0]], rowbuf.at[rb], gsem[rb]).wait()
            pltpu.sync_copy(rowbuf.at[rb], agg_sh.at[ring.at[b, 1]],
                            add=True)

            @pl.when(j + NI < NCH)
            def _():
                pltpu.async_copy(ei_hbm.at[w, j + NI], ring.at[b], isem[b])

            @pl.when(j + NBUF < NCH)
            def _():
                pltpu.make_async_copy(
                    ei_hbm.at[w, j + NBUF], ring.at[sbn], isem[sbn]).wait()
                pltpu.async_copy(
                    g_hbm.at[ring.at[sbn, 0]], rowbuf.at[rb], gsem[rb])
        return carry

    lax.fori_loop(0, NCH // NI, group_body, 0)
    plsc.subcore_barrier()

    pltpu.sync_copy(agg_sh.at[pl.ds(s * RPT, RPT)],
                    pout_hbm.at[c, pl.ds(s * RPT, RPT)])


def _norm_body(degp_ref, f0_ref, srcn_ref, dstn_ref, af0_ref, g0_ref):
    dsrc = degp_ref[0, 0] + degp_ref[1, 0]
    ddst = degp_ref[0, 1] + degp_ref[1, 1]
    srcn = lax.rsqrt(jnp.maximum(dsrc, 1.0))[:, None]
    dstn = lax.rsqrt(jnp.maximum(ddst, 1.0))[:, None]
    f0 = f0_ref[...]
    srcn_b = jnp.broadcast_to(srcn, (N2, D))
    srcn_ref[...] = srcn_b
    dstn_ref[...] = jnp.broadcast_to((1.0 - ALPHA) * dstn, (N2, D))
    af0_ref[...] = ALPHA * f0
    g0_ref[...] = f0 * srcn_b


_norm_call = pl.pallas_call(
    _norm_body,
    out_shape=[
        jax.ShapeDtypeStruct((N2, D), jnp.float32),
        jax.ShapeDtypeStruct((N2, D), jnp.float32),
        jax.ShapeDtypeStruct((N2, D), jnp.float32),
        jax.ShapeDtypeStruct((N2, D), jnp.float32),
    ],
)


_MIXB = 1024


def _mix_body(p0_ref, p1_ref, srcn_ref, dstn_ref, af0_ref, feat_ref, g_ref):
    feat = dstn_ref[...] * (p0_ref[...] + p1_ref[...]) + af0_ref[...]
    feat_ref[...] = feat
    g_ref[...] = feat * srcn_ref[...]


_mix_call = pl.pallas_call(
    _mix_body,
    grid=(N2 // _MIXB,),
    in_specs=[pl.BlockSpec((_MIXB, D), lambda i: (i, 0))] * 5,
    out_specs=[pl.BlockSpec((_MIXB, D), lambda i: (i, 0))] * 2,
    out_shape=[
        jax.ShapeDtypeStruct((N2, D), jnp.float32),
        jax.ShapeDtypeStruct((N2, D), jnp.float32),
    ],
)


@jax.jit
def kernel(features, edge_index):
    pad = jnp.full((2, E2 - E), N, jnp.int32)
    ei = jnp.concatenate([edge_index, pad], axis=1)
    ei4 = ei.reshape(2, NW, NCH, CHUNK).transpose(1, 2, 0, 3)
    zd = jnp.zeros((2, N2), jnp.float32)
    znd = jnp.zeros((N2, D), jnp.float32)
    f0 = jnp.pad(features, ((0, N2 - N), (0, 0)))

    degp = _deg_kernel(ei4, zd)
    srcn_b, dstn_b, af0, g = _norm_call(degp, f0)

    feat = f0
    for _ in range(K):
        p = _prop_kernel(g, ei4, znd)
        feat, g = _mix_call(p[0], p[1], srcn_b, dstn_b, af0)
    return feat[:N]


# speedup vs baseline: 1.9911x; 1.9911x over previous
import functools

import jax
import jax.numpy as jnp
from jax import lax
from jax.experimental import pallas as pl
from jax.experimental.pallas import tpu as pltpu
from jax.experimental.pallas import tpu_sc as plsc

N = 10000
E = 320000
D = 128
K = 10
ALPHA = 0.1

NW = 32
N2 = 10240
CHUNK = 112
NCH = 90
EPW = CHUNK * NCH
E2 = NW * EPW
RPT = N2 // 16
NBUF = 2
NI = 2 * NBUF

_mesh = plsc.VectorSubcoreMesh(core_axis_name="c", subcore_axis_name="s")


@functools.partial(
    pl.kernel,
    mesh=_mesh,
    out_type=jax.ShapeDtypeStruct((2, 2, N2), jnp.float32),
    scratch_types=[
        pltpu.VMEM((NCH, 2, CHUNK), jnp.int32),
        pltpu.VMEM((112,), jnp.float32),
        pltpu.VMEM_SHARED((N2,), jnp.float32),
        pltpu.VMEM_SHARED((N2,), jnp.float32),
        pltpu.SemaphoreType.DMA,
    ],
)
def _deg_kernel(ei_hbm, zd_hbm, degout_hbm,
                idx_v, ones_v, dsrc_sh, ddst_sh, dsem):
    c = lax.axis_index("c")
    s = lax.axis_index("s")
    w = s * 2 + c
    for i in range(7):
        ones_v[pl.ds(i * 16, 16)] = jnp.ones((16,), jnp.float32)
    pltpu.sync_copy(zd_hbm.at[0, pl.ds(s * RPT, RPT)],
                    dsrc_sh.at[pl.ds(s * RPT, RPT)])
    pltpu.sync_copy(zd_hbm.at[1, pl.ds(s * RPT, RPT)],
                    ddst_sh.at[pl.ds(s * RPT, RPT)])
    plsc.subcore_barrier()

    pltpu.sync_copy(ei_hbm.at[w], idx_v)
    ones = ones_v.at[pl.ds(0, CHUNK)]

    def body(j, carry):
        pltpu.sync_copy(ones, dsrc_sh.at[idx_v.at[j, 0]], add=True)
        pltpu.sync_copy(ones, ddst_sh.at[idx_v.at[j, 1]], add=True)
        return carry

    lax.fori_loop(0, NCH, body, 0)
    plsc.subcore_barrier()

    pltpu.sync_copy(dsrc_sh.at[pl.ds(s * RPT, RPT)],
                    degout_hbm.at[c, 0, pl.ds(s * RPT, RPT)])
    pltpu.sync_copy(ddst_sh.at[pl.ds(s * RPT, RPT)],
                    degout_hbm.at[c, 1, pl.ds(s * RPT, RPT)])


@functools.partial(
    pl.kernel,
    mesh=_mesh,
    out_type=jax.ShapeDtypeStruct((2, N2, D), jnp.float32),
    scratch_types=[
        pltpu.VMEM((NI, 2, CHUNK), jnp.int32),
        pltpu.VMEM((NBUF, CHUNK, D), jnp.float32),
        pltpu.VMEM_SHARED((N2, D), jnp.float32),
    ]
    + [pltpu.SemaphoreType.DMA] * (NI + NBUF),
)
def _prop_kernel(g_hbm, ei_hbm, znd_hbm, pout_hbm,
                 ring, rowbuf, agg_sh, *sems):
    isem = sems[:NI]
    gsem = sems[NI:]
    c = lax.axis_index("c")
    s = lax.axis_index("s")
    w = s * 2 + c
    for b in range(NI):
        pltpu.async_copy(ei_hbm.at[w, b], ring.at[b], isem[b])
    pltpu.sync_copy(znd_hbm.at[pl.ds(s * RPT, RPT)],
                    agg_sh.at[pl.ds(s * RPT, RPT)])
    plsc.subcore_barrier()

    for b in range(NBUF):
        pltpu.make_async_copy(ei_hbm.at[w, b], ring.at[b], isem[b]).wait()
        pltpu.async_copy(g_hbm.at[ring.at[b, 0]], rowbuf.at[b], gsem[b])

    def group_body(gi, carry):
        for b in range(NI):
            j = gi * NI + b
            rb = b % NBUF
            sbn = (b + NBUF) % NI
            pltpu.make_async_copy(
                g_hbm.at[ring.at[b, 0]], rowbuf.at[rb], gsem[rb]).wait()
            pltpu.sync_copy(rowbuf.at[rb], agg_sh.at[ring.at[b, 1]],
                            add=True)

            @pl.when(j + NI < NCH)
            def _():
                pltpu.async_copy(ei_hbm.at[w, j + NI], ring.at[b], isem[b])

            @pl.when(j + NBUF < NCH)
            def _():
                pltpu.make_async_copy(
                    ei_hbm.at[w, j + NBUF], ring.at[sbn], isem[sbn]).wait()
                pltpu.async_copy(
                    g_hbm.at[ring.at[sbn, 0]], rowbuf.at[rb], gsem[rb])
        return carry

    lax.fori_loop(0, NCH // NI, group_body, 0)
    plsc.subcore_barrier()

    pltpu.sync_copy(agg_sh.at[pl.ds(s * RPT, RPT)],
                    pout_hbm.at[c, pl.ds(s * RPT, RPT)])


def _norm_body(degp_ref, f0_ref, srcn_ref, dstn_ref, af0_ref, g0_ref):
    dsrc = degp_ref[0, 0] + degp_ref[1, 0]
    ddst = degp_ref[0, 1] + degp_ref[1, 1]
    srcn = lax.rsqrt(jnp.maximum(dsrc, 1.0))[:, None]
    dstn = lax.rsqrt(jnp.maximum(ddst, 1.0))[:, None]
    f0 = f0_ref[...]
    srcn_b = jnp.broadcast_to(srcn, (N2, D))
    srcn_ref[...] = srcn_b
    dstn_ref[...] = jnp.broadcast_to((1.0 - ALPHA) * dstn, (N2, D))
    af0_ref[...] = ALPHA * f0
    g0_ref[...] = f0 * srcn_b


_norm_call = pl.pallas_call(
    _norm_body,
    out_shape=[
        jax.ShapeDtypeStruct((N2, D), jnp.float32),
        jax.ShapeDtypeStruct((N2, D), jnp.float32),
        jax.ShapeDtypeStruct((N2, D), jnp.float32),
        jax.ShapeDtypeStruct((N2, D), jnp.float32),
    ],
)


_MIXB = 1024


def _mix_body(p0_ref, p1_ref, srcn_ref, dstn_ref, af0_ref, feat_ref, g_ref):
    feat = dstn_ref[...] * (p0_ref[...] + p1_ref[...]) + af0_ref[...]
    feat_ref[...] = feat
    g_ref[...] = feat * srcn_ref[...]


_mix_call = pl.pallas_call(
    _mix_body,
    grid=(N2 // _MIXB,),
    in_specs=[pl.BlockSpec((_MIXB, D), lambda i: (i, 0))] * 5,
    out_specs=[pl.BlockSpec((_MIXB, D), lambda i: (i, 0))] * 2,
    out_shape=[
        jax.ShapeDtypeStruct((N2, D), jnp.float32),
        jax.ShapeDtypeStruct((N2, D), jnp.float32),
    ],
)


@jax.jit
def kernel(features, edge_index):
    pad = jnp.full((2, E2 - E), N, jnp.int32)
    ei = jnp.concatenate([edge_index, pad], axis=1)
    ei4 = ei.reshape(2, NW, NCH, CHUNK).transpose(1, 2, 0, 3)
    zd = jnp.zeros((2, N2), jnp.float32)
    znd = jnp.zeros((N2, D), jnp.float32)
    f0 = jnp.pad(features, ((0, N2 - N), (0, 0)))

    degp = _deg_kernel(ei4, zd)
    srcn_b, dstn_b, af0, g = _norm_call(degp, f0)

    feat = f0
    for _ in range(K):
        p = _prop_kernel(g, ei4, znd)
        feat, g = _mix_call(p[0], p[1], srcn_b, dstn_b, af0)
    return feat[:N]
